# baseline (speedup 1.0000x reference)
import functools

import jax
import jax.numpy as jnp
from jax import lax
from jax.experimental import pallas as pl
from jax.experimental.pallas import tpu as pltpu
from jax.experimental.pallas import tpu_sc as plsc

N = 16384
F = 1024
H1 = 512
H2 = 512
E = 8

BLK = 256
NB = 72
NPAD = NB * BLK

NC = 2
NS = 16
NW = NC * NS
ROWS_PER_W = N // NW
CHUNK = 32
NCHUNK = ROWS_PER_W // CHUNK


def _routing(species):
    s = species.astype(jnp.int32)
    eye = jnp.arange(E, dtype=jnp.int32)
    onehot = (s[:, None] == eye[None, :]).astype(jnp.int32)
    cum = jnp.cumsum(onehot, axis=0)
    counts = cum[-1]
    rank = jnp.sum(cum * onehot, axis=1) - 1
    padded = ((counts + BLK - 1) // BLK) * BLK
    pad_starts = jnp.concatenate(
        [jnp.zeros((1,), jnp.int32), jnp.cumsum(padded)[:-1].astype(jnp.int32)])
    dst = jnp.sum(onehot * pad_starts[None, :], axis=1) + rank
    b_idx = jnp.arange(NB, dtype=jnp.int32)
    bs = pad_starts // BLK
    be = (pad_starts + padded) // BLK
    in_reg = (b_idx[:, None] >= bs[None, :]) & (b_idx[:, None] < be[None, :])
    block_expert = jnp.sum(jnp.where(in_reg, eye[None, :], 0), axis=1)
    return dst.astype(jnp.int32), block_expert.astype(jnp.int32)


def _sc_scatter_rows(features, dst):
    mesh = plsc.VectorSubcoreMesh(core_axis_name="c", subcore_axis_name="s")
    dst3 = dst.reshape(NW, NCHUNK, CHUNK)

    @functools.partial(
        pl.kernel,
        out_type=jax.ShapeDtypeStruct((NPAD, F), jnp.float32),
        mesh=mesh,
        scratch_types=[
            pltpu.VMEM((NCHUNK, CHUNK), jnp.int32),
            pltpu.VMEM((2, CHUNK, F), jnp.float32),
            pltpu.SemaphoreType.DMA,
            pltpu.SemaphoreType.DMA,
            pltpu.SemaphoreType.DMA,
            pltpu.SemaphoreType.DMA,
        ],
    )
    def k(feat_hbm, dst_hbm, out_hbm, idx_v, rows_v, si0, si1, so0, so1):
        wid = lax.axis_index("s") * NC + lax.axis_index("c")
        sin = (si0, si1)
        sout = (so0, so1)
        pltpu.sync_copy(dst_hbm.at[wid], idx_v)

        def start_in(c):
            base = wid * ROWS_PER_W + c * CHUNK
            return pltpu.async_copy(
                feat_hbm.at[pl.ds(base, CHUNK)], rows_v.at[c % 2], sin[c % 2])

        def start_out(c):
            return pltpu.async_copy(
                rows_v.at[c % 2], out_hbm.at[idx_v.at[c]], sout[c % 2])

        cps_in = [None] * NCHUNK
        cps_out = [None] * NCHUNK
        cps_in[0] = start_in(0)
        for c in range(NCHUNK):
            cps_in[c].wait()
            cps_out[c] = start_out(c)
            if c + 1 < NCHUNK:
                if c - 1 >= 0:
                    cps_out[c - 1].wait()
                cps_in[c + 1] = start_in(c + 1)
        cps_out[NCHUNK - 2].wait()
        cps_out[NCHUNK - 1].wait()

    return k(features, dst3)


def _half_chain(x, w1, b1, w2, b2, w3, b3):
    xb = x.astype(jnp.bfloat16)
    h = jnp.tanh(jnp.dot(xb, w1, preferred_element_type=jnp.float32) + b1)
    h = jnp.tanh(
        jnp.dot(h.astype(jnp.bfloat16), w2, preferred_element_type=jnp.float32)
        + b2)
    return jnp.sum(h * w3, axis=1, keepdims=True) + b3


def _mlp_body(eid_ref, x_ref, w1_ref, b1_ref, w2_ref, b2_ref, w3_ref, b3_ref,
              out_ref):
    w1 = w1_ref[0]
    w2 = w2_ref[0]
    b1 = b1_ref[0]
    b2 = b2_ref[0]
    w3 = w3_ref[0]
    b3 = b3_ref[0]
    Hh = BLK // 2
    ea = _half_chain(x_ref[0:Hh], w1, b1, w2, b2, w3, b3)
    eb = _half_chain(x_ref[Hh:BLK], w1, b1, w2, b2, w3, b3)
    out_ref[0, 0:Hh] = ea
    out_ref[0, Hh:BLK] = eb


def _tc_grouped_mlp(block_expert, xs, W1, b1, W2, b2, W3, b3, nb=NB):
    w1b = W1.astype(jnp.bfloat16)
    w2b = W2.astype(jnp.bfloat16)
    b1r = b1.reshape(E, 1, H1)
    b2r = b2.reshape(E, 1, H2)
    w3r = W3.reshape(E, H2).reshape(E, 1, H2)
    b3r = b3.reshape(E, 1, 1)
    grid_spec = pltpu.PrefetchScalarGridSpec(
        num_scalar_prefetch=1,
        grid=(nb,),
        in_specs=[
            pl.BlockSpec((BLK, F), lambda i, eid: (i, 0)),
            pl.BlockSpec((1, F, H1), lambda i, eid: (eid[i], 0, 0)),
            pl.BlockSpec((1, 1, H1), lambda i, eid: (eid[i], 0, 0)),
            pl.BlockSpec((1, H1, H2), lambda i, eid: (eid[i], 0, 0)),
            pl.BlockSpec((1, 1, H2), lambda i, eid: (eid[i], 0, 0)),
            pl.BlockSpec((1, 1, H2), lambda i, eid: (eid[i], 0, 0)),
            pl.BlockSpec((1, 1, 1), lambda i, eid: (eid[i], 0, 0)),
        ],
        out_specs=pl.BlockSpec((1, BLK, 1), lambda i, eid: (i, 0, 0)),
    )
    out = pl.pallas_call(
        _mlp_body,
        grid_spec=grid_spec,
        out_shape=jax.ShapeDtypeStruct((nb, BLK, 1), jnp.float32),
    )(block_expert, xs, w1b, b1r, w2b, b2r, w3r, b3r)
    return out.reshape(nb * BLK)


def _sc_gather_out(e_pad, dst):
    mesh = plsc.VectorSubcoreMesh(core_axis_name="c", subcore_axis_name="s")

    @functools.partial(
        pl.kernel,
        out_type=jax.ShapeDtypeStruct((N,), jnp.float32),
        mesh=mesh,
        scratch_types=[
            pltpu.VMEM((NPAD,), jnp.float32),
            pltpu.VMEM((ROWS_PER_W,), jnp.int32),
            pltpu.VMEM((ROWS_PER_W,), jnp.float32),
        ],
        compiler_params=pltpu.CompilerParams(needs_layout_passes=False),
    )
    def k(e_hbm, dst_hbm, out_hbm, etab_v, idx_v, out_v):
        wid = lax.axis_index("s") * NC + lax.axis_index("c")
        base = wid * ROWS_PER_W
        pltpu.sync_copy(e_hbm, etab_v)
        pltpu.sync_copy(dst_hbm.at[pl.ds(base, ROWS_PER_W)], idx_v)
        for j in range(ROWS_PER_W // 16):
            idxs = idx_v[pl.ds(j * 16, 16)]
            out_v[pl.ds(j * 16, 16)] = plsc.load_gather(etab_v, [idxs])
        pltpu.sync_copy(out_v, out_hbm.at[pl.ds(base, ROWS_PER_W)])

    return k(e_pad, dst)


def kernel(features, species_indices, W1, b1, W2, b2, W3, b3):
    dst, block_expert = _routing(species_indices)
    xs = _sc_scatter_rows(features, dst)
    e_pad = _tc_grouped_mlp(block_expert, xs, W1, b1, W2, b2, W3, b3)
    return _sc_gather_out(e_pad, dst)


# speedup vs baseline: 1.0244x; 1.0244x over previous
import functools

import jax
import jax.numpy as jnp
from jax import lax
from jax.experimental import pallas as pl
from jax.experimental.pallas import tpu as pltpu
from jax.experimental.pallas import tpu_sc as plsc

N = 16384
F = 1024
H1 = 512
H2 = 512
E = 8

BLK = 256
NB = 72
NPAD = NB * BLK

NC = 2
NS = 16
NW = NC * NS
ROWS_PER_W = N // NW
CHUNK = 64
NCHUNK = ROWS_PER_W // CHUNK


def _routing(species):
    s = species.astype(jnp.int32)
    eye = jnp.arange(E, dtype=jnp.int32)
    onehot = (s[:, None] == eye[None, :]).astype(jnp.int32)
    cum = jnp.cumsum(onehot, axis=0)
    counts = cum[-1]
    rank = jnp.sum(cum * onehot, axis=1) - 1
    padded = ((counts + BLK - 1) // BLK) * BLK
    pad_starts = jnp.concatenate(
        [jnp.zeros((1,), jnp.int32), jnp.cumsum(padded)[:-1].astype(jnp.int32)])
    dst = jnp.sum(onehot * pad_starts[None, :], axis=1) + rank
    b_idx = jnp.arange(NB, dtype=jnp.int32)
    bs = pad_starts // BLK
    be = (pad_starts + padded) // BLK
    in_reg = (b_idx[:, None] >= bs[None, :]) & (b_idx[:, None] < be[None, :])
    block_expert = jnp.sum(jnp.where(in_reg, eye[None, :], 0), axis=1)
    return dst.astype(jnp.int32), block_expert.astype(jnp.int32)


def _sc_scatter_rows(features, dst):
    mesh = plsc.VectorSubcoreMesh(core_axis_name="c", subcore_axis_name="s")
    dst3 = dst.reshape(NW, NCHUNK, CHUNK)

    @functools.partial(
        pl.kernel,
        out_type=jax.ShapeDtypeStruct((NPAD, F), jnp.float32),
        mesh=mesh,
        scratch_types=[
            pltpu.VMEM((NCHUNK, CHUNK), jnp.int32),
            pltpu.VMEM((CHUNK, F), jnp.float32),
            pltpu.SemaphoreType.DMA,
        ],
    )
    def k(feat_hbm, dst_hbm, out_hbm, idx_v, rows_v, sem):
        wid = lax.axis_index("s") * NC + lax.axis_index("c")
        pltpu.sync_copy(dst_hbm.at[wid], idx_v)
        for c in range(NCHUNK):
            base = wid * ROWS_PER_W + c * CHUNK
            pltpu.sync_copy(feat_hbm.at[pl.ds(base, CHUNK)], rows_v)
            pltpu.async_copy(rows_v, out_hbm.at[idx_v.at[c]], sem).wait()

    return k(features, dst3)


def _mlp_body(eid_ref, x_ref, w1_ref, b1_ref, w2_ref, b2_ref, w3_ref, b3_ref,
              out_ref):
    x = x_ref[...].astype(jnp.bfloat16)
    h = jnp.tanh(
        jnp.dot(x, w1_ref[0], preferred_element_type=jnp.float32) + b1_ref[0])
    h = jnp.tanh(
        jnp.dot(h.astype(jnp.bfloat16), w2_ref[0],
                preferred_element_type=jnp.float32) + b2_ref[0])
    e = jnp.sum(h * w3_ref[0], axis=1, keepdims=True) + b3_ref[0]
    out_ref[0] = e


def _tc_grouped_mlp(block_expert, xs, W1, b1, W2, b2, W3, b3, nb=NB):
    w1b = W1.astype(jnp.bfloat16)
    w2b = W2.astype(jnp.bfloat16)
    b1r = b1.reshape(E, 1, H1)
    b2r = b2.reshape(E, 1, H2)
    w3r = W3.reshape(E, H2).reshape(E, 1, H2)
    b3r = b3.reshape(E, 1, 1)
    grid_spec = pltpu.PrefetchScalarGridSpec(
        num_scalar_prefetch=1,
        grid=(nb,),
        in_specs=[
            pl.BlockSpec((BLK, F), lambda i, eid: (i, 0)),
            pl.BlockSpec((1, F, H1), lambda i, eid: (eid[i], 0, 0)),
            pl.BlockSpec((1, 1, H1), lambda i, eid: (eid[i], 0, 0)),
            pl.BlockSpec((1, H1, H2), lambda i, eid: (eid[i], 0, 0)),
            pl.BlockSpec((1, 1, H2), lambda i, eid: (eid[i], 0, 0)),
            pl.BlockSpec((1, 1, H2), lambda i, eid: (eid[i], 0, 0)),
            pl.BlockSpec((1, 1, 1), lambda i, eid: (eid[i], 0, 0)),
        ],
        out_specs=pl.BlockSpec((1, BLK, 1), lambda i, eid: (i, 0, 0)),
    )
    out = pl.pallas_call(
        _mlp_body,
        grid_spec=grid_spec,
        out_shape=jax.ShapeDtypeStruct((nb, BLK, 1), jnp.float32),
    )(block_expert, xs, w1b, b1r, w2b, b2r, w3r, b3r)
    return out.reshape(nb * BLK)


def _sc_gather_out(e_pad, dst):
    mesh = plsc.VectorSubcoreMesh(core_axis_name="c", subcore_axis_name="s")

    @functools.partial(
        pl.kernel,
        out_type=jax.ShapeDtypeStruct((N,), jnp.float32),
        mesh=mesh,
        scratch_types=[
            pltpu.VMEM((NPAD,), jnp.float32),
            pltpu.VMEM((ROWS_PER_W,), jnp.int32),
            pltpu.VMEM((ROWS_PER_W,), jnp.float32),
        ],
        compiler_params=pltpu.CompilerParams(needs_layout_passes=False),
    )
    def k(e_hbm, dst_hbm, out_hbm, etab_v, idx_v, out_v):
        wid = lax.axis_index("s") * NC + lax.axis_index("c")
        base = wid * ROWS_PER_W
        pltpu.sync_copy(e_hbm, etab_v)
        pltpu.sync_copy(dst_hbm.at[pl.ds(base, ROWS_PER_W)], idx_v)
        for j in range(ROWS_PER_W // 16):
            idxs = idx_v[pl.ds(j * 16, 16)]
            out_v[pl.ds(j * 16, 16)] = plsc.load_gather(etab_v, [idxs])
        pltpu.sync_copy(out_v, out_hbm.at[pl.ds(base, ROWS_PER_W)])

    return k(e_pad, dst)


def kernel(features, species_indices, W1, b1, W2, b2, W3, b3):
    dst, block_expert = _routing(species_indices)
    xs = _sc_scatter_rows(features, dst)
    e_pad = _tc_grouped_mlp(block_expert, xs, W1, b1, W2, b2, W3, b3)
    return _sc_gather_out(e_pad, dst)
